# baseline (speedup 1.0000x reference)
import functools

import jax
import jax.numpy as jnp
from jax import lax
from jax.experimental import pallas as pl
from jax.experimental.pallas import tpu as pltpu
from jax.experimental.pallas import tpu_sc as plsc

_H = 32
_W = 32
_E = 768
_BINS = 16
_IMG = 512
_PATCH = 16
_T = _H * _W

_BB = 4
_NCHUNK = 4
_NC = 2
_NS = 16
_NW = _NC * _NS


def _pool_sc_body(chunk_off, per_w, depth_hbm, out_hbm, buf, rs_buf, outbuf):
    wid = lax.axis_index("s") * _NC + lax.axis_index("c")
    base = chunk_off + wid * per_w
    pltpu.sync_copy(depth_hbm.at[pl.ds(base, per_w)], buf)

    lane = lax.iota(jnp.int32, 16)

    def strip_body(i, carry):
        for g in range(_W):
            acc = buf[i, 0, pl.ds(g * _PATCH, 16)]
            for j in range(1, _PATCH):
                acc = acc + buf[i, j, pl.ds(g * _PATCH, 16)]
            rs_buf[pl.ds(g * _PATCH, 16)] = acc
        for half in range(2):
            idx0 = lane * _PATCH + half * 16 * _PATCH
            tot = plsc.load_gather(rs_buf, [idx0])
            for k in range(1, _PATCH):
                tot = tot + plsc.load_gather(rs_buf, [idx0 + k])
            outbuf[i, pl.ds(half * 16, 16)] = tot
        return carry

    lax.fori_loop(0, per_w, strip_body, 0)
    pltpu.sync_copy(outbuf, out_hbm.at[pl.ds(wid * per_w, per_w), :])


def _pool_sc(depth_strips, chunk_off, n_rows):
    per_w = n_rows // _NW
    mesh = plsc.VectorSubcoreMesh(
        core_axis_name="c", subcore_axis_name="s", num_cores=_NC,
        num_subcores=_NS)
    return pl.kernel(
        functools.partial(_pool_sc_body, chunk_off, per_w),
        out_type=jax.ShapeDtypeStruct((n_rows, _W), jnp.float32),
        mesh=mesh,
        scratch_types=[
            pltpu.VMEM((per_w, _PATCH, _IMG), jnp.float32),
            pltpu.VMEM((_IMG,), jnp.float32),
            pltpu.VMEM((per_w, _W), jnp.float32),
        ],
        compiler_params=pltpu.CompilerParams(needs_layout_passes=False),
    )(depth_strips)


def _tc_body(pt_ref, pool_ref, row_ref, col_ref, demb_ref, out_ref, rc_ref):
    b = pl.program_id(0)

    @pl.when(b == 0)
    def _():
        row = row_ref[...]
        col = col_ref[...]
        rc = row[:, None, :] + col[None, :, :]
        rc_ref[...] = rc.reshape(_T, _E)

    t0 = lax.broadcasted_iota(jnp.int32, (_T, _H), 0)
    t1 = lax.broadcasted_iota(jnp.int32, (_T, _H), 1)
    onehot_r = jnp.where(t0 // _W == t1, 1.0, 0.0)
    onehot_c = jnp.where(t0 % _W == t1, 1.0, 0.0)
    k = lax.broadcasted_iota(jnp.int32, (_T, _BINS), 1)

    for j in range(_BB):
        pooled = pool_ref[j] * (1.0 / (_PATCH * _PATCH))
        dpos = jnp.sqrt(jnp.clip(pooled, 0.0, 1.0)) * (_BINS - 1)

        rowsel = jnp.dot(onehot_r, dpos)
        dpos_col = jnp.sum(rowsel * onehot_c, axis=1, keepdims=True)

        lo_f = jnp.floor(dpos_col)
        alpha = dpos_col - lo_f
        lo = lo_f.astype(jnp.int32)
        hi = jnp.minimum(lo + 1, _BINS - 1)
        w = jnp.where(k == lo, 1.0 - alpha, 0.0) + jnp.where(k == hi, alpha, 0.0)
        depth_pe = jnp.dot(w, demb_ref[...])

        out_ref[j] = pt_ref[j] + rc_ref[...] + depth_pe


def _tc_chunk(carry, patch_tokens, pooled, row_emb, col_emb, depth_emb,
              chunk_idx, cb):
    bsz = patch_tokens.shape[0]
    nsteps = cb // _BB
    off = chunk_idx * nsteps
    body = _tc_body

    specs = [
        pl.BlockSpec((_BB, _T, _E), lambda b: (off + b, 0, 0)),
        pl.BlockSpec((_BB, _H, _W), lambda b: (b, 0, 0)),
        pl.BlockSpec((_H, _E), lambda b: (0, 0)),
        pl.BlockSpec((_W, _E), lambda b: (0, 0)),
        pl.BlockSpec((_BINS, _E), lambda b: (0, 0)),
    ]
    args = [patch_tokens, pooled, row_emb, col_emb, depth_emb]
    aliases = {}
    if carry is not None:
        def body_c(carry_ref, *refs):
            _tc_body(*refs)
        body = body_c
        specs = [pl.BlockSpec(memory_space=pl.ANY)] + specs
        args = [carry] + args
        aliases = {0: 0}
    return pl.pallas_call(
        body,
        grid=(nsteps,),
        in_specs=specs,
        out_specs=pl.BlockSpec((_BB, _T, _E), lambda b: (off + b, 0, 0)),
        out_shape=jax.ShapeDtypeStruct((bsz, _T, _E), jnp.float32),
        scratch_shapes=[pltpu.VMEM((_T, _E), jnp.float32)],
        input_output_aliases=aliases,
        compiler_params=pltpu.CompilerParams(
            dimension_semantics=("arbitrary",),
            vmem_limit_bytes=100 * 1024 * 1024,
        ),
    )(*args)


def kernel(patch_tokens, depth_ch, row_emb, col_emb, depth_emb):
    bsz = patch_tokens.shape[0]
    strips = depth_ch.reshape(bsz * _H, _PATCH, _IMG)
    cb = bsz // _NCHUNK
    rows_per_chunk = cb * _H
    sums = [
        _pool_sc(strips, c * rows_per_chunk, rows_per_chunk)
        for c in range(_NCHUNK)
    ]
    out = None
    for c in range(_NCHUNK):
        pooled = sums[c].reshape(cb, _H, _W)
        out = _tc_chunk(out, patch_tokens, pooled, row_emb, col_emb,
                        depth_emb, c, cb)
    return out


# speedup vs baseline: 1.4196x; 1.4196x over previous
import jax
import jax.numpy as jnp
from jax import lax
from jax.experimental import pallas as pl
from jax.experimental.pallas import tpu as pltpu

_H = 32
_W = 32
_E = 768
_BINS = 16
_IMG = 512
_PATCH = 16
_T = _H * _W


_BB = 4


def _body(pt_ref, d_ref, row_ref, col_ref, demb_ref, out_ref, rc_ref):
    b = pl.program_id(0)

    @pl.when(b == 0)
    def _():
        row = row_ref[...]
        col = col_ref[...]
        rc = row[:, None, :] + col[None, :, :]
        rc_ref[...] = rc.reshape(_T, _E)

    a0 = lax.broadcasted_iota(jnp.int32, (_H, _IMG), 0)
    a1 = lax.broadcasted_iota(jnp.int32, (_H, _IMG), 1)
    p1 = jnp.where(a1 // _PATCH == a0, 1.0 / _PATCH, 0.0)
    b0 = lax.broadcasted_iota(jnp.int32, (_IMG, _W), 0)
    b1 = lax.broadcasted_iota(jnp.int32, (_IMG, _W), 1)
    p2 = jnp.where(b0 // _PATCH == b1, 1.0 / _PATCH, 0.0)

    t0 = lax.broadcasted_iota(jnp.int32, (_T, _H), 0)
    t1 = lax.broadcasted_iota(jnp.int32, (_T, _H), 1)
    onehot_r = jnp.where(t0 // _W == t1, 1.0, 0.0)
    onehot_c = jnp.where(t0 % _W == t1, 1.0, 0.0)
    k = lax.broadcasted_iota(jnp.int32, (_T, _BINS), 1)

    for j in range(_BB):
        d = d_ref[j, 0]
        pooled = jnp.dot(jnp.dot(p1, d), p2)
        dpos = jnp.sqrt(jnp.clip(pooled, 0.0, 1.0)) * (_BINS - 1)

        rowsel = jnp.dot(onehot_r, dpos)
        dpos_col = jnp.sum(rowsel * onehot_c, axis=1, keepdims=True)

        lo_f = jnp.floor(dpos_col)
        alpha = dpos_col - lo_f
        lo = lo_f.astype(jnp.int32)
        hi = jnp.minimum(lo + 1, _BINS - 1)
        w = jnp.where(k == lo, 1.0 - alpha, 0.0) + jnp.where(k == hi, alpha, 0.0)
        depth_pe = jnp.dot(w, demb_ref[...])

        out_ref[j] = pt_ref[j] + rc_ref[...] + depth_pe


def kernel(patch_tokens, depth_ch, row_emb, col_emb, depth_emb):
    bsz = patch_tokens.shape[0]
    return pl.pallas_call(
        _body,
        grid=(bsz // _BB,),
        in_specs=[
            pl.BlockSpec((_BB, _T, _E), lambda b: (b, 0, 0)),
            pl.BlockSpec((_BB, 1, _IMG, _IMG), lambda b: (b, 0, 0, 0)),
            pl.BlockSpec((_H, _E), lambda b: (0, 0)),
            pl.BlockSpec((_W, _E), lambda b: (0, 0)),
            pl.BlockSpec((_BINS, _E), lambda b: (0, 0)),
        ],
        out_specs=pl.BlockSpec((_BB, _T, _E), lambda b: (b, 0, 0)),
        out_shape=jax.ShapeDtypeStruct((bsz, _T, _E), jnp.float32),
        scratch_shapes=[pltpu.VMEM((_T, _E), jnp.float32)],
        compiler_params=pltpu.CompilerParams(
            dimension_semantics=("arbitrary",),
            vmem_limit_bytes=100 * 1024 * 1024,
        ),
    )(patch_tokens, depth_ch, row_emb, col_emb, depth_emb)
